# Initial kernel scaffold; baseline (speedup 1.0000x reference)
import jax
import jax.numpy as jnp
from jax.experimental import pallas as pl


def kernel(x, edge_index, batch, Wr0, Wa0, b0, g0, be0, Wr1, Wa1, b1, g1, be1, Ws1, bs1, Ws2, bs2, Wh1, bh1, Wh2, bh2, Wh3, bh3):
    raise NotImplementedError("write your pallas kernel here")


# speedup vs baseline: 4.8659x; 4.8659x over previous
import jax
import jax.numpy as jnp
from jax import lax
from jax.experimental import pallas as pl
from jax.experimental.pallas import tpu as pltpu
from jax.experimental.pallas import tpu_sc as plsc

N_NODES = 10000
N_EDGES = 320000
D_FEAT = 128
BATCH_SIZE = 100
NODES_PER_GRAPH = 100

NC = 2
NS = 16
NW = NC * NS
EDGES_PER_TILE = N_EDGES // NW
CHUNK = 80
N_CHUNKS = EDGES_PER_TILE // CHUNK
ROW_STRIDE = 624
ROW_COPY = 640

_MESH = plsc.VectorSubcoreMesh(core_axis_name="c", subcore_axis_name="s")


def _sc_agg_body(h_hbm, src_hbm, dst_hbm, zeros_nd, out_agg,
                 src_v, dst_v, rows_v, agg_sp, sem):
    c = lax.axis_index("c")
    s = lax.axis_index("s")
    wid = s * NC + c

    pltpu.sync_copy(zeros_nd.at[pl.ds(s * ROW_STRIDE, ROW_COPY)],
                    agg_sp.at[pl.ds(s * ROW_STRIDE, ROW_COPY)])
    plsc.subcore_barrier()

    base = wid * EDGES_PER_TILE

    def chunk_body(i, carry):
        off = base + i * CHUNK
        pltpu.sync_copy(src_hbm.at[pl.ds(off, CHUNK)], src_v)
        pltpu.sync_copy(dst_hbm.at[pl.ds(off, CHUNK)], dst_v)
        pltpu.async_copy(h_hbm.at[src_v], rows_v, sem).wait()
        pltpu.sync_copy(rows_v, agg_sp.at[dst_v], add=True)
        return carry

    lax.fori_loop(0, N_CHUNKS, chunk_body, None)

    plsc.subcore_barrier()
    pltpu.sync_copy(agg_sp.at[pl.ds(s * ROW_STRIDE, ROW_COPY)],
                    out_agg.at[pl.ds(c * N_NODES + s * ROW_STRIDE, ROW_COPY)])


_sc_agg = pl.kernel(
    _sc_agg_body,
    out_type=jax.ShapeDtypeStruct((NC * N_NODES, D_FEAT), jnp.float32),
    mesh=_MESH,
    scratch_types=[
        pltpu.VMEM((CHUNK,), jnp.int32),
        pltpu.VMEM((CHUNK,), jnp.int32),
        pltpu.VMEM((CHUNK, D_FEAT), jnp.float32),
        pltpu.VMEM_SHARED((N_NODES, D_FEAT), jnp.float32),
        pltpu.SemaphoreType.DMA,
    ],
)


def _sc_deg_body(dst_hbm, zeros_nd, ones_c, out_deg,
                 dst_v, ones_v, deg_sp, sem):
    c = lax.axis_index("c")
    s = lax.axis_index("s")
    wid = s * NC + c

    pltpu.sync_copy(zeros_nd.at[pl.ds(s * ROW_STRIDE, ROW_COPY)],
                    deg_sp.at[pl.ds(s * ROW_STRIDE, ROW_COPY)])
    pltpu.sync_copy(ones_c, ones_v)
    plsc.subcore_barrier()

    base = wid * EDGES_PER_TILE

    def chunk_body(i, carry):
        off = base + i * CHUNK
        pltpu.sync_copy(dst_hbm.at[pl.ds(off, CHUNK)], dst_v)
        pltpu.sync_copy(ones_v, deg_sp.at[dst_v], add=True)
        return carry

    lax.fori_loop(0, N_CHUNKS, chunk_body, None)

    plsc.subcore_barrier()
    pltpu.sync_copy(deg_sp.at[pl.ds(s * ROW_STRIDE, ROW_COPY)],
                    out_deg.at[pl.ds(c * N_NODES + s * ROW_STRIDE, ROW_COPY)])


_sc_deg = pl.kernel(
    _sc_deg_body,
    out_type=jax.ShapeDtypeStruct((NC * N_NODES, D_FEAT), jnp.float32),
    mesh=_MESH,
    scratch_types=[
        pltpu.VMEM((CHUNK,), jnp.int32),
        pltpu.VMEM((CHUNK, D_FEAT), jnp.float32),
        pltpu.VMEM_SHARED((N_NODES, D_FEAT), jnp.float32),
        pltpu.SemaphoreType.DMA,
    ],
)


def _tc_layer_body(h_ref, part_ref, deg_ref, wr_ref, wa_ref, b_ref, g_ref,
                   be_ref, out_ref):
    agg = part_ref[:N_NODES] + part_ref[N_NODES:]
    deg = jnp.maximum(deg_ref[:N_NODES] + deg_ref[N_NODES:], 1.0)
    mean = agg / deg
    y = (jnp.dot(h_ref[...], wr_ref[...], preferred_element_type=jnp.float32)
         + jnp.dot(mean, wa_ref[...], preferred_element_type=jnp.float32)
         + b_ref[...])
    mu = jnp.mean(y, axis=0, keepdims=True)
    var = jnp.mean((y - mu) ** 2, axis=0, keepdims=True)
    out_ref[...] = jnp.maximum((y - mu) / jnp.sqrt(var + 1e-5) * g_ref[...]
                               + be_ref[...], 0.0)


_tc_layer = pl.pallas_call(
    _tc_layer_body,
    out_shape=jax.ShapeDtypeStruct((N_NODES, D_FEAT), jnp.float32),
)


def _tc_final_body(h_ref, part_ref, deg_ref, wr_ref, wa_ref, b_ref, g_ref,
                   be_ref, pool_ref, ws1_ref, bs1_ref, ws2_ref, bs2_ref,
                   wh1_ref, bh1_ref, wh2_ref, bh2_ref, wh3_ref, bh3_ref,
                   out_ref):
    agg = part_ref[:N_NODES] + part_ref[N_NODES:]
    deg = jnp.maximum(deg_ref[:N_NODES] + deg_ref[N_NODES:], 1.0)
    mean = agg / deg
    y = (jnp.dot(h_ref[...], wr_ref[...], preferred_element_type=jnp.float32)
         + jnp.dot(mean, wa_ref[...], preferred_element_type=jnp.float32)
         + b_ref[...])
    mu = jnp.mean(y, axis=0, keepdims=True)
    var = jnp.mean((y - mu) ** 2, axis=0, keepdims=True)
    h2 = jnp.maximum((y - mu) / jnp.sqrt(var + 1e-5) * g_ref[...]
                     + be_ref[...], 0.0)
    xg = jnp.dot(pool_ref[...], h2, preferred_element_type=jnp.float32)
    t = jnp.maximum(xg, 0.0)
    t = jnp.dot(t, ws1_ref[...], preferred_element_type=jnp.float32) + bs1_ref[...]
    t = jnp.dot(t, ws2_ref[...], preferred_element_type=jnp.float32) + bs2_ref[...]
    t = jnp.maximum(t, 0.0)
    t = jnp.maximum(jnp.dot(t, wh1_ref[...], preferred_element_type=jnp.float32)
                    + bh1_ref[...], 0.0)
    t = jnp.maximum(jnp.dot(t, wh2_ref[...], preferred_element_type=jnp.float32)
                    + bh2_ref[...], 0.0)
    out_ref[...] = (jnp.dot(t, wh3_ref[...], preferred_element_type=jnp.float32)
                    + bh3_ref[...])


_tc_final = pl.pallas_call(
    _tc_final_body,
    out_shape=jax.ShapeDtypeStruct((BATCH_SIZE, 1), jnp.float32),
)


def kernel(x, edge_index, batch, Wr0, Wa0, b0, g0, be0, Wr1, Wa1, b1, g1,
           be1, Ws1, bs1, Ws2, bs2, Wh1, bh1, Wh2, bh2, Wh3, bh3):
    src = edge_index[0]
    dst = edge_index[1]
    zeros_nd = jnp.zeros((N_NODES, D_FEAT), jnp.float32)
    ones_c = jnp.ones((CHUNK, D_FEAT), jnp.float32)

    deg2 = _sc_deg(dst, zeros_nd, ones_c)[:, :1]
    agg0 = _sc_agg(x, src, dst, zeros_nd)

    h1 = _tc_layer(x, agg0, deg2, Wr0, Wa0, b0, g0, be0)

    agg1 = _sc_agg(h1, src, dst, zeros_nd)

    pool = (batch[None, :] == jnp.arange(BATCH_SIZE, dtype=batch.dtype)[:, None]
            ).astype(jnp.float32) * (1.0 / NODES_PER_GRAPH)

    return _tc_final(h1, agg1, deg2, Wr1, Wa1, b1, g1, be1, pool,
                     Ws1, bs1, Ws2, bs2, Wh1, bh1, Wh2, bh2, Wh3, bh3)


# speedup vs baseline: 8.6225x; 1.7720x over previous
import jax
import jax.numpy as jnp
from jax import lax
from jax.experimental import pallas as pl
from jax.experimental.pallas import tpu as pltpu
from jax.experimental.pallas import tpu_sc as plsc

N_NODES = 10000
N_EDGES = 320000
D_FEAT = 128
BATCH_SIZE = 100
NODES_PER_GRAPH = 100

NC = 2
NS = 16
NW = NC * NS
EDGES_PER_TILE = N_EDGES // NW
CHUNK = 80
N_CHUNKS = EDGES_PER_TILE // CHUNK
ROW_STRIDE = 624
ROW_COPY = 640
K_AGG = 2
K_DEG = 5

_MESH = plsc.VectorSubcoreMesh(core_axis_name="c", subcore_axis_name="s")


def _sc_agg_body(h_hbm, src_hbm, dst_hbm, zeros_nd, out_agg,
                 src_slab, dst_slab, rows, gsems, ssems, agg_sp):
    c = lax.axis_index("c")
    s = lax.axis_index("s")
    wid = s * NC + c

    pltpu.sync_copy(zeros_nd.at[pl.ds(s * ROW_STRIDE, ROW_COPY)],
                    agg_sp.at[pl.ds(s * ROW_STRIDE, ROW_COPY)])
    pltpu.sync_copy(src_hbm.at[pl.ds(wid * EDGES_PER_TILE, EDGES_PER_TILE)],
                    src_slab)
    pltpu.sync_copy(dst_hbm.at[wid], dst_slab)
    plsc.subcore_barrier()

    def group_body(g, carry):
        gathers = []
        for j in range(K_AGG):
            i = g * K_AGG + j
            gathers.append(pltpu.async_copy(
                h_hbm.at[src_slab.at[pl.ds(i * CHUNK, CHUNK)]],
                rows[j], gsems[j]))
        scatters = []
        for j in range(K_AGG):
            i = g * K_AGG + j
            gathers[j].wait()
            scatters.append(pltpu.async_copy(rows[j],
                                             agg_sp.at[dst_slab.at[i]],
                                             ssems[j], add=True))
        for d in scatters:
            d.wait()
        return carry

    n_groups = N_CHUNKS // K_AGG
    lax.fori_loop(0, n_groups, group_body, None)
    for i in range(n_groups * K_AGG, N_CHUNKS):
        pltpu.async_copy(h_hbm.at[src_slab.at[pl.ds(i * CHUNK, CHUNK)]],
                         rows[0], gsems[0]).wait()
        pltpu.async_copy(rows[0], agg_sp.at[dst_slab.at[i]],
                         ssems[0], add=True).wait()

    plsc.subcore_barrier()
    pltpu.sync_copy(agg_sp.at[pl.ds(s * ROW_STRIDE, ROW_COPY)],
                    out_agg.at[pl.ds(c * N_NODES + s * ROW_STRIDE, ROW_COPY)])


_sc_agg = pl.kernel(
    _sc_agg_body,
    out_type=jax.ShapeDtypeStruct((NC * N_NODES, D_FEAT), jnp.float32),
    mesh=_MESH,
    scratch_types=[
        pltpu.VMEM((EDGES_PER_TILE,), jnp.int32),
        pltpu.VMEM((N_CHUNKS, CHUNK), jnp.int32),
        [pltpu.VMEM((CHUNK, D_FEAT), jnp.float32) for _ in range(K_AGG)],
        [pltpu.SemaphoreType.DMA for _ in range(K_AGG)],
        [pltpu.SemaphoreType.DMA for _ in range(K_AGG)],
        pltpu.VMEM_SHARED((N_NODES, D_FEAT), jnp.float32),
    ],
)


def _sc_deg_body(dst_hbm, zeros_nd, ones_c, out_deg,
                 dst_slab, ones_v, ssems, deg_sp):
    c = lax.axis_index("c")
    s = lax.axis_index("s")
    wid = s * NC + c

    pltpu.sync_copy(zeros_nd.at[pl.ds(s * ROW_STRIDE, ROW_COPY)],
                    deg_sp.at[pl.ds(s * ROW_STRIDE, ROW_COPY)])
    pltpu.sync_copy(ones_c, ones_v)
    pltpu.sync_copy(dst_hbm.at[wid], dst_slab)
    plsc.subcore_barrier()

    def group_body(g, carry):
        scatters = []
        for j in range(K_DEG):
            i = g * K_DEG + j
            scatters.append(pltpu.async_copy(ones_v,
                                             deg_sp.at[dst_slab.at[i]],
                                             ssems[j], add=True))
        for d in scatters:
            d.wait()
        return carry

    lax.fori_loop(0, N_CHUNKS // K_DEG, group_body, None)

    plsc.subcore_barrier()
    pltpu.sync_copy(deg_sp.at[pl.ds(s * ROW_STRIDE, ROW_COPY)],
                    out_deg.at[pl.ds(c * N_NODES + s * ROW_STRIDE, ROW_COPY)])


_sc_deg = pl.kernel(
    _sc_deg_body,
    out_type=jax.ShapeDtypeStruct((NC * N_NODES, D_FEAT), jnp.float32),
    mesh=_MESH,
    scratch_types=[
        pltpu.VMEM((N_CHUNKS, CHUNK), jnp.int32),
        pltpu.VMEM((CHUNK, D_FEAT), jnp.float32),
        [pltpu.SemaphoreType.DMA for _ in range(K_DEG)],
        pltpu.VMEM_SHARED((N_NODES, D_FEAT), jnp.float32),
    ],
)


def _tc_layer_body(h_ref, part_ref, deg_ref, wr_ref, wa_ref, b_ref, g_ref,
                   be_ref, out_ref):
    agg = part_ref[:N_NODES] + part_ref[N_NODES:]
    deg = jnp.maximum(deg_ref[:N_NODES] + deg_ref[N_NODES:], 1.0)
    mean = agg / deg
    y = (jnp.dot(h_ref[...], wr_ref[...], preferred_element_type=jnp.float32)
         + jnp.dot(mean, wa_ref[...], preferred_element_type=jnp.float32)
         + b_ref[...])
    mu = jnp.mean(y, axis=0, keepdims=True)
    var = jnp.mean((y - mu) ** 2, axis=0, keepdims=True)
    out_ref[...] = jnp.maximum((y - mu) / jnp.sqrt(var + 1e-5) * g_ref[...]
                               + be_ref[...], 0.0)


_tc_layer = pl.pallas_call(
    _tc_layer_body,
    out_shape=jax.ShapeDtypeStruct((N_NODES, D_FEAT), jnp.float32),
)


def _tc_final_body(h_ref, part_ref, deg_ref, wr_ref, wa_ref, b_ref, g_ref,
                   be_ref, pool_ref, ws1_ref, bs1_ref, ws2_ref, bs2_ref,
                   wh1_ref, bh1_ref, wh2_ref, bh2_ref, wh3_ref, bh3_ref,
                   out_ref):
    agg = part_ref[:N_NODES] + part_ref[N_NODES:]
    deg = jnp.maximum(deg_ref[:N_NODES] + deg_ref[N_NODES:], 1.0)
    mean = agg / deg
    y = (jnp.dot(h_ref[...], wr_ref[...], preferred_element_type=jnp.float32)
         + jnp.dot(mean, wa_ref[...], preferred_element_type=jnp.float32)
         + b_ref[...])
    mu = jnp.mean(y, axis=0, keepdims=True)
    var = jnp.mean((y - mu) ** 2, axis=0, keepdims=True)
    h2 = jnp.maximum((y - mu) / jnp.sqrt(var + 1e-5) * g_ref[...]
                     + be_ref[...], 0.0)
    xg = jnp.dot(pool_ref[...], h2, preferred_element_type=jnp.float32)
    t = jnp.maximum(xg, 0.0)
    t = jnp.dot(t, ws1_ref[...], preferred_element_type=jnp.float32) + bs1_ref[...]
    t = jnp.dot(t, ws2_ref[...], preferred_element_type=jnp.float32) + bs2_ref[...]
    t = jnp.maximum(t, 0.0)
    t = jnp.maximum(jnp.dot(t, wh1_ref[...], preferred_element_type=jnp.float32)
                    + bh1_ref[...], 0.0)
    t = jnp.maximum(jnp.dot(t, wh2_ref[...], preferred_element_type=jnp.float32)
                    + bh2_ref[...], 0.0)
    out_ref[...] = (jnp.dot(t, wh3_ref[...], preferred_element_type=jnp.float32)
                    + bh3_ref[...])


_tc_final = pl.pallas_call(
    _tc_final_body,
    out_shape=jax.ShapeDtypeStruct((BATCH_SIZE, 1), jnp.float32),
)


def kernel(x, edge_index, batch, Wr0, Wa0, b0, g0, be0, Wr1, Wa1, b1, g1,
           be1, Ws1, bs1, Ws2, bs2, Wh1, bh1, Wh2, bh2, Wh3, bh3):
    src_flat = edge_index[0]
    dst3 = edge_index[1].reshape(NW, N_CHUNKS, CHUNK)
    zeros_nd = jnp.zeros((N_NODES, D_FEAT), jnp.float32)
    ones_c = jnp.ones((CHUNK, D_FEAT), jnp.float32)

    deg2 = _sc_deg(dst3, zeros_nd, ones_c)[:, :1]
    agg0 = _sc_agg(x, src_flat, dst3, zeros_nd)

    h1 = _tc_layer(x, agg0, deg2, Wr0, Wa0, b0, g0, be0)

    agg1 = _sc_agg(h1, src_flat, dst3, zeros_nd)

    pool = (batch[None, :] == jnp.arange(BATCH_SIZE, dtype=batch.dtype)[:, None]
            ).astype(jnp.float32) * (1.0 / NODES_PER_GRAPH)

    return _tc_final(h1, agg1, deg2, Wr1, Wa1, b1, g1, be1, pool,
                     Ws1, bs1, Ws2, bs2, Wh1, bh1, Wh2, bh2, Wh3, bh3)


# speedup vs baseline: 8.7477x; 1.0145x over previous
import jax
import jax.numpy as jnp
from jax import lax
from jax.experimental import pallas as pl
from jax.experimental.pallas import tpu as pltpu
from jax.experimental.pallas import tpu_sc as plsc

N_NODES = 10000
N_EDGES = 320000
D_FEAT = 128
BATCH_SIZE = 100
NODES_PER_GRAPH = 100

NC = 2
NS = 16
NW = NC * NS
EDGES_PER_TILE = N_EDGES // NW
CHUNK = 80
N_CHUNKS = EDGES_PER_TILE // CHUNK
ROW_STRIDE = 624
ROW_COPY = 640
K_AGG = 2
K_DEG = 5

_MESH = plsc.VectorSubcoreMesh(core_axis_name="c", subcore_axis_name="s")


def _sc_agg_body(h_hbm, src_hbm, dst_hbm, zeros_nd, out_agg,
                 src_slab, dst_slab, rows, gsems, ssems, agg_sp):
    c = lax.axis_index("c")
    s = lax.axis_index("s")
    wid = s * NC + c

    pltpu.sync_copy(zeros_nd.at[pl.ds(s * ROW_STRIDE, ROW_COPY)],
                    agg_sp.at[pl.ds(s * ROW_STRIDE, ROW_COPY)])
    pltpu.sync_copy(src_hbm.at[pl.ds(wid * EDGES_PER_TILE, EDGES_PER_TILE)],
                    src_slab)
    pltpu.sync_copy(dst_hbm.at[wid], dst_slab)
    plsc.subcore_barrier()

    n_groups = N_CHUNKS // K_AGG

    def group_body(g, carry):
        gathers = []
        for j in range(K_AGG):
            i = g * K_AGG + j

            @pl.when(g > 0)
            def _(j=j, i=i):
                pltpu.make_async_copy(
                    rows[j], agg_sp.at[dst_slab.at[i - K_AGG]],
                    ssems[j]).wait()

            gathers.append(pltpu.async_copy(
                h_hbm.at[src_slab.at[pl.ds(i * CHUNK, CHUNK)]],
                rows[j], gsems[j]))
        for j in range(K_AGG):
            i = g * K_AGG + j
            gathers[j].wait()
            pltpu.async_copy(rows[j], agg_sp.at[dst_slab.at[i]],
                             ssems[j], add=True)
        return carry

    lax.fori_loop(0, n_groups, group_body, None)
    for j in range(K_AGG):
        i = (n_groups - 1) * K_AGG + j
        pltpu.make_async_copy(rows[j], agg_sp.at[dst_slab.at[i]],
                              ssems[j]).wait()
    for i in range(n_groups * K_AGG, N_CHUNKS):
        pltpu.async_copy(h_hbm.at[src_slab.at[pl.ds(i * CHUNK, CHUNK)]],
                         rows[0], gsems[0]).wait()
        pltpu.async_copy(rows[0], agg_sp.at[dst_slab.at[i]],
                         ssems[0], add=True).wait()

    plsc.subcore_barrier()
    pltpu.sync_copy(agg_sp.at[pl.ds(s * ROW_STRIDE, ROW_COPY)],
                    out_agg.at[pl.ds(c * N_NODES + s * ROW_STRIDE, ROW_COPY)])


_sc_agg = pl.kernel(
    _sc_agg_body,
    out_type=jax.ShapeDtypeStruct((NC * N_NODES, D_FEAT), jnp.float32),
    mesh=_MESH,
    scratch_types=[
        pltpu.VMEM((EDGES_PER_TILE,), jnp.int32),
        pltpu.VMEM((N_CHUNKS, CHUNK), jnp.int32),
        [pltpu.VMEM((CHUNK, D_FEAT), jnp.float32) for _ in range(K_AGG)],
        [pltpu.SemaphoreType.DMA for _ in range(K_AGG)],
        [pltpu.SemaphoreType.DMA for _ in range(K_AGG)],
        pltpu.VMEM_SHARED((N_NODES, D_FEAT), jnp.float32),
    ],
)


def _sc_deg_body(dst_hbm, zeros_nd, ones_c, out_deg,
                 dst_slab, ones_v, ssems, deg_sp):
    c = lax.axis_index("c")
    s = lax.axis_index("s")
    wid = s * NC + c

    pltpu.sync_copy(zeros_nd.at[pl.ds(s * ROW_STRIDE, ROW_COPY)],
                    deg_sp.at[pl.ds(s * ROW_STRIDE, ROW_COPY)])
    pltpu.sync_copy(ones_c, ones_v)
    pltpu.sync_copy(dst_hbm.at[wid], dst_slab)
    plsc.subcore_barrier()

    def group_body(g, carry):
        scatters = []
        for j in range(K_DEG):
            i = g * K_DEG + j
            scatters.append(pltpu.async_copy(ones_v,
                                             deg_sp.at[dst_slab.at[i]],
                                             ssems[j], add=True))
        for d in scatters:
            d.wait()
        return carry

    lax.fori_loop(0, N_CHUNKS // K_DEG, group_body, None)

    plsc.subcore_barrier()
    pltpu.sync_copy(deg_sp.at[pl.ds(s * ROW_STRIDE, ROW_COPY)],
                    out_deg.at[pl.ds(c * N_NODES + s * ROW_STRIDE, ROW_COPY)])


_sc_deg = pl.kernel(
    _sc_deg_body,
    out_type=jax.ShapeDtypeStruct((NC * N_NODES, D_FEAT), jnp.float32),
    mesh=_MESH,
    scratch_types=[
        pltpu.VMEM((N_CHUNKS, CHUNK), jnp.int32),
        pltpu.VMEM((CHUNK, D_FEAT), jnp.float32),
        [pltpu.SemaphoreType.DMA for _ in range(K_DEG)],
        pltpu.VMEM_SHARED((N_NODES, D_FEAT), jnp.float32),
    ],
)


def _tc_layer_body(h_ref, part_ref, deg_ref, wr_ref, wa_ref, b_ref, g_ref,
                   be_ref, out_ref):
    agg = part_ref[:N_NODES] + part_ref[N_NODES:]
    deg = jnp.maximum(deg_ref[:N_NODES] + deg_ref[N_NODES:], 1.0)
    mean = agg / deg
    y = (jnp.dot(h_ref[...], wr_ref[...], preferred_element_type=jnp.float32)
         + jnp.dot(mean, wa_ref[...], preferred_element_type=jnp.float32)
         + b_ref[...])
    mu = jnp.mean(y, axis=0, keepdims=True)
    var = jnp.mean((y - mu) ** 2, axis=0, keepdims=True)
    out_ref[...] = jnp.maximum((y - mu) / jnp.sqrt(var + 1e-5) * g_ref[...]
                               + be_ref[...], 0.0)


_tc_layer = pl.pallas_call(
    _tc_layer_body,
    out_shape=jax.ShapeDtypeStruct((N_NODES, D_FEAT), jnp.float32),
)


def _tc_final_body(h_ref, part_ref, deg_ref, wr_ref, wa_ref, b_ref, g_ref,
                   be_ref, pool_ref, ws1_ref, bs1_ref, ws2_ref, bs2_ref,
                   wh1_ref, bh1_ref, wh2_ref, bh2_ref, wh3_ref, bh3_ref,
                   out_ref):
    agg = part_ref[:N_NODES] + part_ref[N_NODES:]
    deg = jnp.maximum(deg_ref[:N_NODES] + deg_ref[N_NODES:], 1.0)
    mean = agg / deg
    y = (jnp.dot(h_ref[...], wr_ref[...], preferred_element_type=jnp.float32)
         + jnp.dot(mean, wa_ref[...], preferred_element_type=jnp.float32)
         + b_ref[...])
    mu = jnp.mean(y, axis=0, keepdims=True)
    var = jnp.mean((y - mu) ** 2, axis=0, keepdims=True)
    h2 = jnp.maximum((y - mu) / jnp.sqrt(var + 1e-5) * g_ref[...]
                     + be_ref[...], 0.0)
    xg = jnp.dot(pool_ref[...], h2, preferred_element_type=jnp.float32)
    t = jnp.maximum(xg, 0.0)
    t = jnp.dot(t, ws1_ref[...], preferred_element_type=jnp.float32) + bs1_ref[...]
    t = jnp.dot(t, ws2_ref[...], preferred_element_type=jnp.float32) + bs2_ref[...]
    t = jnp.maximum(t, 0.0)
    t = jnp.maximum(jnp.dot(t, wh1_ref[...], preferred_element_type=jnp.float32)
                    + bh1_ref[...], 0.0)
    t = jnp.maximum(jnp.dot(t, wh2_ref[...], preferred_element_type=jnp.float32)
                    + bh2_ref[...], 0.0)
    out_ref[...] = (jnp.dot(t, wh3_ref[...], preferred_element_type=jnp.float32)
                    + bh3_ref[...])


_tc_final = pl.pallas_call(
    _tc_final_body,
    out_shape=jax.ShapeDtypeStruct((BATCH_SIZE, 1), jnp.float32),
)


def kernel(x, edge_index, batch, Wr0, Wa0, b0, g0, be0, Wr1, Wa1, b1, g1,
           be1, Ws1, bs1, Ws2, bs2, Wh1, bh1, Wh2, bh2, Wh3, bh3):
    src_flat = edge_index[0]
    dst3 = edge_index[1].reshape(NW, N_CHUNKS, CHUNK)
    zeros_nd = jnp.zeros((N_NODES, D_FEAT), jnp.float32)
    ones_c = jnp.ones((CHUNK, D_FEAT), jnp.float32)

    deg2 = _sc_deg(dst3, zeros_nd, ones_c)[:, :1]
    agg0 = _sc_agg(x, src_flat, dst3, zeros_nd)

    h1 = _tc_layer(x, agg0, deg2, Wr0, Wa0, b0, g0, be0)

    agg1 = _sc_agg(h1, src_flat, dst3, zeros_nd)

    pool = (batch[None, :] == jnp.arange(BATCH_SIZE, dtype=batch.dtype)[:, None]
            ).astype(jnp.float32) * (1.0 / NODES_PER_GRAPH)

    return _tc_final(h1, agg1, deg2, Wr1, Wa1, b1, g1, be1, pool,
                     Ws1, bs1, Ws2, bs2, Wh1, bh1, Wh2, bh2, Wh3, bh3)
